# baseline (speedup 1.0000x reference)
import functools

import jax
import jax.numpy as jnp
from jax import lax
from jax.experimental import pallas as pl
from jax.experimental.pallas import tpu as pltpu
from jax.experimental.pallas import tpu_sc as plsc

_INCREASE_DELTA = 0.05
_ABS_THRESHOLD = 0.8

_B = 16
_S = 4096
_L = 16
_NCHUNK = _S // _L
_NGROUP = _NCHUNK // _L
_PAD = _L
_NEG = -3e38
_POS = 3e38


def _seg_body(ent_hbm, seg_hbm, pem_hbm, fb_hbm,
              row_v, seg_v, pem_v, fb_v,
              sums_v, lasts_v, cpref_v, mpref_v):
    wid = lax.axis_index("s")

    rb = wid * _S
    row_v[pl.ds(0, _L)] = jnp.full((_L,), _NEG, jnp.float32)
    pltpu.sync_copy(ent_hbm.at[pl.ds(rb, _S)], row_v.at[pl.ds(_PAD, _S)])
    row_v[pl.ds(_PAD + _S, _L)] = jnp.full((_L,), _POS, jnp.float32)
    lasts_v[pl.ds(0, _L)] = jnp.zeros((_L,), jnp.int32)

    lane = lax.iota(jnp.int32, _L)

    @functools.partial(plsc.parallel_loop, 0, _NCHUNK, unroll=4)
    def chunk_a(i):
        base = _PAD + i * _L
        prev = row_v[pl.ds(base - 1, _L)]
        e = row_v[pl.ds(base, _L)]
        nxt = row_v[pl.ds(base + 1, _L)]
        inc = (e > prev + _INCREASE_DELTA) | (e > _ABS_THRESHOLD)
        pem = (nxt > e + _INCREASE_DELTA) | (nxt > _ABS_THRESHOLD)
        inc_i = inc.astype(jnp.int32)
        off = i * _L
        seg_v[pl.ds(off, _L)] = plsc.cumsum(inc_i)
        pem_v[pl.ds(off, _L)] = pem.astype(jnp.int32)
        fp = jnp.where(inc, off + lane, 0)
        fb_v[pl.ds(off, _L)] = plsc.cummax(fp)
        cnt = plsc.all_reduce_population_count(inc)
        sums_v[pl.ds(off, _L)] = cnt
        ffs = plsc.all_reduce_ffs(lax.rev(inc_i, (0,)) != 0)
        lastpos = jnp.where(cnt > 0, (off + 15) - ffs, 0)
        lasts_v[pl.ds(off + _L, _L)] = lastpos

    def group_b(j, carry):
        carry_sum, carry_max = carry
        idx = j * (_L * _L) + lane * _L
        cnts = plsc.load_gather(sums_v, [idx])
        exl = plsc.load_gather(lasts_v, [idx])
        lst = plsc.load_gather(lasts_v, [idx + _L])
        lcs = plsc.cumsum(cnts)
        cpref_v[pl.ds(j * _L, _L)] = (lcs - cnts) + carry_sum
        lcm = plsc.cummax(exl)
        mvec = jnp.maximum(lcm, carry_max)
        mpref_v[pl.ds(j * _L, _L)] = mvec
        new_sum = carry_sum + jnp.max(lcs)
        new_max = jnp.maximum(carry_max, jnp.max(lst))
        return new_sum, new_max

    lax.fori_loop(
        0, _NGROUP, group_b,
        (jnp.zeros((_L,), jnp.int32), jnp.zeros((_L,), jnp.int32)),
    )

    @functools.partial(plsc.parallel_loop, 0, _NCHUNK, unroll=4)
    def chunk_c(i):
        off = i * _L
        isplat = jnp.full((_L,), i, jnp.int32)
        s_off = plsc.load_gather(cpref_v, [isplat])
        m = plsc.load_gather(mpref_v, [isplat])
        seg_v[pl.ds(off, _L)] = seg_v[pl.ds(off, _L)] + (s_off - 1)
        fb_v[pl.ds(off, _L)] = jnp.maximum(fb_v[pl.ds(off, _L)], m)

    pltpu.sync_copy(seg_v, seg_hbm.at[pl.ds(rb, _S)])
    pltpu.sync_copy(pem_v, pem_hbm.at[pl.ds(rb, _S)])
    pltpu.sync_copy(fb_v, fb_hbm.at[pl.ds(rb, _S)])


@jax.jit
def _segmenter(entropy_bits):
    mesh = plsc.VectorSubcoreMesh(
        core_axis_name="c", subcore_axis_name="s", num_cores=1, num_subcores=16
    )
    out = jax.ShapeDtypeStruct((_B * _S,), jnp.int32)
    run = functools.partial(
        pl.kernel,
        out_type=(out, out, out),
        mesh=mesh,
        compiler_params=pltpu.CompilerParams(
            needs_layout_passes=False, skip_device_barrier=True
        ),
        scratch_types=[
            pltpu.VMEM((_PAD + _S + _L,), jnp.float32),
            pltpu.VMEM((_S,), jnp.int32),
            pltpu.VMEM((_S,), jnp.int32),
            pltpu.VMEM((_S,), jnp.int32),
            pltpu.VMEM((_NCHUNK * _L,), jnp.int32),
            pltpu.VMEM(((_NCHUNK + 1) * _L,), jnp.int32),
            pltpu.VMEM((_NCHUNK,), jnp.int32),
            pltpu.VMEM((_NCHUNK,), jnp.int32),
        ],
    )(_seg_body)
    seg, pem, fb = run(entropy_bits.reshape(_B * _S))
    return (
        seg.reshape(_B, _S),
        pem.reshape(_B, _S) != 0,
        fb.reshape(_B, _S),
    )


def kernel(entropy_bits):
    return _segmenter(entropy_bits)
